# Initial kernel scaffold; baseline (speedup 1.0000x reference)
import jax
import jax.numpy as jnp
from jax.experimental import pallas as pl


def kernel(input_bitarrays, hidden_indices, final_indices, hidden_invert, final_invert, output_shape):
    raise NotImplementedError("write your pallas kernel here")


# speedup vs baseline: 1.8022x; 1.8022x over previous
import functools

import jax
import jax.numpy as jnp
from jax import lax
from jax.experimental import pallas as pl
from jax.experimental.pallas import tpu as pltpu
from jax.experimental.pallas import tpu_sc as plsc

NUM_INPUTS = 1024
NUM_OUTPUTS = 1024
NUM_LAYERS = 8
NPL = 16384
W = 1024

WC = 16
NCHUNK = W // WC
NC = 2
NS = 16
CHUNKS_PER_CORE = NCHUNK // NC
GPT = NPL // NS
FGPT = NUM_OUTPUTS // NS
NJ = (2 * GPT) // 128
IRT = NUM_INPUTS // NS


def _sc_nand(in_r, hidx_r, hinv_r, fidx_r, finv_r):
    mesh = plsc.VectorSubcoreMesh(core_axis_name="c", subcore_axis_name="s")

    @functools.partial(
        pl.kernel,
        out_type=jax.ShapeDtypeStruct((NS * NCHUNK * FGPT, WC), jnp.int32),
        mesh=mesh,
        compiler_params=pltpu.CompilerParams(use_tc_tiling_on_sc=False),
        scratch_types=dict(
            a_sh=pltpu.VMEM_SHARED((NPL, WC), jnp.int32),
            b_sh=pltpu.VMEM_SHARED((NPL, WC), jnp.int32),
            idx_v=pltpu.VMEM((NUM_LAYERS, NJ, 128), jnp.int32),
            inv_v=pltpu.VMEM((NUM_LAYERS, GPT), jnp.int32),
            fidx_v=pltpu.VMEM((2 * FGPT,), jnp.int32),
            finv_v=pltpu.VMEM((FGPT,), jnp.int32),
            rows_v=pltpu.VMEM((2 * GPT, WC), jnp.int32),
            out_v=pltpu.VMEM((GPT, WC), jnp.int32),
            stage_v=pltpu.VMEM((IRT, WC), jnp.int32),
            fout_v=pltpu.VMEM((FGPT, WC), jnp.int32),
            sem=pltpu.SemaphoreType.DMA,
        ),
    )
    def k(in_hbm, hidx_hbm, hinv_hbm, fidx_hbm, finv_hbm, out_hbm, *,
          a_sh, b_sh, idx_v, inv_v, fidx_v, finv_v, rows_v, out_v,
          stage_v, fout_v, sem):
        cid = lax.axis_index("c")
        tid = lax.axis_index("s")

        pltpu.sync_copy(hidx_hbm.at[pl.ds(tid * NUM_LAYERS, NUM_LAYERS)],
                        idx_v)
        pltpu.sync_copy(hinv_hbm.at[pl.ds(tid * NUM_LAYERS, NUM_LAYERS)],
                        inv_v)
        pltpu.sync_copy(fidx_hbm.at[pl.ds(tid * 2 * FGPT, 2 * FGPT)], fidx_v)
        pltpu.sync_copy(finv_hbm.at[pl.ds(tid * FGPT, FGPT)], finv_v)

        def compute_gates(n_gates, inv_ref, l, dst_ref):
            def grp_body(q, carry):
                if l is not None:
                    m16 = inv_ref[l, pl.ds(16 * q, 16)]
                else:
                    m16 = inv_ref[pl.ds(16 * q, 16)]
                mneg = jnp.int32(0) - m16
                for k in range(16):
                    base = 32 * q + 2 * k
                    a = rows_v[base, :]
                    b = rows_v[base + 1, :]
                    dst_ref[16 * q + k, :] = (a & b) ^ lax.broadcast(
                        mneg[k], (WC,))
                return carry
            lax.fori_loop(0, n_gates // 16, grp_body, 0)

        def chunk_body(i, carry):
            c = cid * CHUNKS_PER_CORE + i
            pltpu.sync_copy(in_hbm.at[pl.ds((tid * NCHUNK + c) * IRT, IRT)], stage_v)
            pltpu.sync_copy(stage_v, a_sh.at[pl.ds(tid * IRT, IRT)])
            plsc.subcore_barrier()

            bufs = (a_sh, b_sh)
            for l in range(NUM_LAYERS):
                src = bufs[l % 2]
                dst = bufs[1 - l % 2]
                handles = [
                    pltpu.async_copy(
                        src.at[idx_v.at[l, j]],
                        rows_v.at[pl.ds(j * 128, 128)],
                        sem,
                    )
                    for j in range(NJ)
                ]
                for h in handles:
                    h.wait()
                compute_gates(GPT, inv_v, l, out_v)
                pltpu.sync_copy(out_v, dst.at[pl.ds(tid * GPT, GPT)])
                plsc.subcore_barrier()

            pltpu.async_copy(
                a_sh.at[fidx_v], rows_v.at[pl.ds(0, 2 * FGPT)], sem
            ).wait()
            compute_gates(FGPT, finv_v, None, fout_v)
            pltpu.sync_copy(fout_v,
                            out_hbm.at[pl.ds((tid * NCHUNK + c) * FGPT, FGPT)])
            plsc.subcore_barrier()
            return carry

        lax.fori_loop(0, CHUNKS_PER_CORE, chunk_body, 0)

    return k(in_r, hidx_r, hinv_r, fidx_r, finv_r)


@jax.jit
def kernel(input_bitarrays, hidden_indices, final_indices, hidden_invert,
           final_invert, output_shape):
    in_r = (input_bitarrays
            .reshape(NS, IRT, NCHUNK, WC)
            .transpose(0, 2, 1, 3)
            .reshape(NS * NCHUNK * IRT, WC))
    hidx_r = (hidden_indices.reshape(NUM_LAYERS, NS, NJ, 128)
              .transpose(1, 0, 2, 3)
              .reshape(NS * NUM_LAYERS, NJ, 128))
    hinv_r = (hidden_invert.astype(jnp.int32)
              .reshape(NUM_LAYERS, NS, GPT)
              .transpose(1, 0, 2)
              .reshape(NS * NUM_LAYERS, GPT))
    fidx_r = final_indices.reshape(NS * 2 * FGPT)
    finv_r = final_invert.astype(jnp.int32).reshape(NS * FGPT)
    out = _sc_nand(in_r, hidx_r, hinv_r, fidx_r, finv_r)
    return (out.reshape(NS, NCHUNK, FGPT, WC)
            .transpose(0, 2, 1, 3)
            .reshape(NUM_OUTPUTS, W))
